# Initial kernel scaffold; baseline (speedup 1.0000x reference)
import functools

import jax
import jax.numpy as jnp
from jax import lax
from jax.experimental import pallas as pl
from jax.experimental.pallas import tpu as pltpu
from jax.experimental.pallas import tpu_sc as plsc

_IN_H = 512
_IN_W = 512
_HW = _IN_H * _IN_W
_O = _HW
_K = 16
_B = 64
_L = 16

_NC = 2
_NS = 16
_NW = _NC * _NS
_OPW = _O // _NW
_T = 32
_STEPS = _OPW // _T
_IDX_ROWS = _T * _K // 128


def _sc_body(table, conn, w, out, connbuf, wbuf, rowsbuf, outbuf, sem):
    wid = lax.axis_index("s") * _NC + lax.axis_index("c")
    o0 = wid * _OPW

    def step(s, carry):
        ob = o0 + s * _T
        pltpu.sync_copy(conn.at[pl.ds(ob * _K // 128, _IDX_ROWS)], connbuf)
        pltpu.sync_copy(w.at[pl.ds(ob * _K, _T * _K)], wbuf)
        cps = [
            pltpu.async_copy(
                table.at[connbuf.at[j]],
                rowsbuf.at[pl.ds(j * 128, 128)],
                sem,
            )
            for j in range(_IDX_ROWS)
        ]
        for c in cps:
            c.wait()

        def one_out(o, carry2):
            accs = [jnp.zeros((_L,), jnp.float32) for _ in range(_B // _L)]
            for k in range(_K):
                r = o * _K + k
                wsp = plsc.load_gather(
                    wbuf, [jnp.full((_L,), r, jnp.int32)]
                )
                for d in range(_B // _L):
                    x = rowsbuf[r, pl.ds(d * _L, _L)]
                    accs[d] = accs[d] + x * wsp
            for d in range(_B // _L):
                outbuf[pl.ds(o * _B + d * _L, _L)] = accs[d]
            return carry2

        lax.fori_loop(0, _T, one_out, 0)
        pltpu.sync_copy(outbuf, out.at[pl.ds(ob * _B, _T * _B)])
        return carry

    lax.fori_loop(0, _STEPS, step, 0)


def kernel(input, connections, weights):
    table = input.reshape(_B, _HW).T
    conn = connections.reshape(_O * _K // 128, 128)
    w = weights.reshape(_O * _K)

    mesh = plsc.VectorSubcoreMesh(core_axis_name="c", subcore_axis_name="s")
    fn = functools.partial(
        pl.kernel,
        mesh=mesh,
        out_type=jax.ShapeDtypeStruct((_O * _B,), jnp.float32),
        scratch_types=[
            pltpu.VMEM((_IDX_ROWS, 128), jnp.int32),
            pltpu.VMEM((_T * _K,), jnp.float32),
            pltpu.VMEM((_T * _K, _B), jnp.float32),
            pltpu.VMEM((_T * _B,), jnp.float32),
            pltpu.SemaphoreType.DMA,
        ],
    )(_sc_body)
    flat = fn(table, conn, w)
    return flat.reshape(_O, _B).T.reshape(_B, _IN_H, _IN_W)


# speedup vs baseline: 3.1121x; 3.1121x over previous
import functools

import jax
import jax.numpy as jnp
from jax import lax
from jax.experimental import pallas as pl
from jax.experimental.pallas import tpu as pltpu
from jax.experimental.pallas import tpu_sc as plsc

_IN_H = 512
_IN_W = 512
_HW = _IN_H * _IN_W
_O = _HW
_K = 16
_B = 64
_L = 16

_NC = 2
_NS = 16
_NW = _NC * _NS
_OPW = _O // _NW
_T = 64
_STEPS = _OPW // _T
_IDX_ROWS = _T * _K // 128


def _sc_body(table, conn, w, out, connbuf, wbuf, rowsbuf, outbuf, sem):
    wid = lax.axis_index("s") * _NC + lax.axis_index("c")
    o0 = wid * _OPW

    def step(s, carry):
        ob = o0 + s * _T
        row0 = pl.multiple_of(ob * _K // 128, _IDX_ROWS)
        pltpu.sync_copy(conn.at[pl.ds(row0, _IDX_ROWS)], connbuf)
        pltpu.sync_copy(w.at[pl.ds(ob * _K, _T * _K)], wbuf)
        cps = [
            pltpu.async_copy(
                table.at[connbuf.at[j]],
                rowsbuf.at[pl.ds(j * 128, 128)],
                sem,
            )
            for j in range(_IDX_ROWS)
        ]
        for c in cps:
            c.wait()

        def one_out(o, carry2):
            accs = [jnp.zeros((_L,), jnp.float32) for _ in range(_B // _L)]
            for k in range(_K):
                r = o * _K + k
                wsp = plsc.load_gather(
                    wbuf, [jnp.full((_L,), r, jnp.int32)]
                )
                for d in range(_B // _L):
                    x = rowsbuf[r, pl.ds(d * _L, _L)]
                    accs[d] = accs[d] + x * wsp
            for d in range(_B // _L):
                outbuf[pl.ds(o * _B + d * _L, _L)] = accs[d]
            return carry2

        lax.fori_loop(0, _T, one_out, 0)
        pltpu.sync_copy(outbuf, out.at[pl.ds(ob * _B, _T * _B)])
        return carry

    lax.fori_loop(0, _STEPS, step, 0)


def kernel(input, connections, weights):
    table = input.reshape(_B, _HW).T
    conn = connections.reshape(_O * _K // 128, 128)
    w = weights.reshape(_O * _K)

    mesh = plsc.VectorSubcoreMesh(core_axis_name="c", subcore_axis_name="s")
    fn = functools.partial(
        pl.kernel,
        mesh=mesh,
        out_type=jax.ShapeDtypeStruct((_O * _B,), jnp.float32),
        scratch_types=[
            pltpu.VMEM((_IDX_ROWS, 128), jnp.int32),
            pltpu.VMEM((_T * _K,), jnp.float32),
            pltpu.VMEM((_T * _K, _B), jnp.float32),
            pltpu.VMEM((_T * _B,), jnp.float32),
            pltpu.SemaphoreType.DMA,
        ],
        compiler_params=pltpu.CompilerParams(
            use_tc_tiling_on_sc=False,
            needs_layout_passes=False,
        ),
    )(_sc_body)
    flat = fn(table, conn, w)
    return flat.reshape(_O, _B).T.reshape(_B, _IN_H, _IN_W)


# speedup vs baseline: 4.1085x; 1.3202x over previous
import functools

import jax
import jax.numpy as jnp
from jax import lax
from jax.experimental import pallas as pl
from jax.experimental.pallas import tpu as pltpu
from jax.experimental.pallas import tpu_sc as plsc

_IN_H = 512
_IN_W = 512
_HW = _IN_H * _IN_W
_O = _HW
_K = 16
_B = 64
_L = 16

_NC = 2
_NS = 16
_NW = _NC * _NS
_OPW = _O // _NW
_T = 32
_STEPS = _OPW // _T
_R = _T * _K
_NG = _R // 128


def _sc_body(table, conn, w, out, connbufs, wbufs, rowsbufs, outbufs,
             gsems, csems, wsems, osems):
    wid = lax.axis_index("s") * _NC + lax.axis_index("c")
    o0 = wid * _OPW

    def issue_gathers(par):
        for j in range(_NG):
            pltpu.async_copy(
                table.at[connbufs[par].at[pl.ds(j * 128, 128)]],
                rowsbufs[par].at[pl.ds(j * 128, 128)],
                gsems[par],
            )

    def start_conn(s, par):
        pltpu.async_copy(conn.at[pl.ds((o0 + s * _T) * _K, _R)],
                         connbufs[par], csems[par])

    def start_w(s, par):
        pltpu.async_copy(w.at[pl.ds((o0 + s * _T) * _K, _R)],
                         wbufs[par], wsems[par])

    def wait_conn(par):
        pltpu.make_async_copy(conn.at[pl.ds(0, _R)], connbufs[par],
                              csems[par]).wait()

    def wait_w(par):
        pltpu.make_async_copy(w.at[pl.ds(0, _R)], wbufs[par],
                              wsems[par]).wait()

    def wait_gathers(par):
        pltpu.make_async_copy(table.at[pl.ds(0, _R)], rowsbufs[par],
                              gsems[par]).wait()

    def wait_out(par):
        pltpu.make_async_copy(out.at[:, pl.ds(0, _T)], outbufs[par],
                              osems[par]).wait()

    pltpu.sync_copy(conn.at[pl.ds(o0 * _K, _R)], connbufs[0])
    issue_gathers(0)
    start_conn(1, 1)
    start_w(0, 0)
    start_w(1, 1)

    viotas = [lax.iota(jnp.int32, _L) + d * _L for d in range(_B // _L)]

    def pair(s2, carry):
        for par in range(2):
            s = s2 * 2 + par
            parn = par ^ 1
            ob = o0 + s * _T

            wait_gathers(par)

            @pl.when(s + 2 < _STEPS)
            def _():
                start_conn(s + 2, par)

            @pl.when(s + 1 < _STEPS)
            def _():
                wait_conn(parn)
                issue_gathers(parn)

            wait_w(par)

            @pl.when(s >= 2)
            def _():
                wait_out(par)

            def one_out(o, carry2):
                wrow = wbufs[par][pl.ds(o * _K, _K)]
                col = jnp.full((_L,), o, jnp.int32)
                accs = [jnp.zeros((_L,), jnp.float32)
                        for _ in range(_B // _L)]
                for k in range(_K):
                    wk = lax.gather(
                        wrow, jnp.full((_L, 1), k, jnp.int32),
                        lax.GatherDimensionNumbers(
                            offset_dims=(), collapsed_slice_dims=(0,),
                            start_index_map=(0,)),
                        slice_sizes=(1,),
                        mode=lax.GatherScatterMode.PROMISE_IN_BOUNDS)
                    for d in range(_B // _L):
                        x = rowsbufs[par][o * _K + k, pl.ds(d * _L, _L)]
                        accs[d] = accs[d] + x * wk
                for d in range(_B // _L):
                    plsc.store_scatter(outbufs[par], [viotas[d], col],
                                       accs[d])
                return carry2

            lax.fori_loop(0, _T, one_out, 0)

            @pl.when(s + 2 < _STEPS)
            def _():
                start_w(s + 2, par)

            pltpu.async_copy(outbufs[par], out.at[:, pl.ds(ob, _T)],
                             osems[par])
        return carry

    lax.fori_loop(0, _STEPS // 2, pair, 0)
    wait_out(0)
    wait_out(1)


def kernel(input, connections, weights):
    table = input.reshape(_B, _HW).T
    conn = connections.reshape(_O * _K)
    w = weights.reshape(_O * _K)

    mesh = plsc.VectorSubcoreMesh(core_axis_name="c", subcore_axis_name="s")
    fn = functools.partial(
        pl.kernel,
        mesh=mesh,
        out_type=jax.ShapeDtypeStruct((_B, _O), jnp.float32),
        scratch_types=[
            [pltpu.VMEM((_R,), jnp.int32) for _ in range(2)],
            [pltpu.VMEM((_R,), jnp.float32) for _ in range(2)],
            [pltpu.VMEM((_R, _B), jnp.float32) for _ in range(2)],
            [pltpu.VMEM((_B, _T), jnp.float32) for _ in range(2)],
            [pltpu.SemaphoreType.DMA for _ in range(2)],
            [pltpu.SemaphoreType.DMA for _ in range(2)],
            [pltpu.SemaphoreType.DMA for _ in range(2)],
            [pltpu.SemaphoreType.DMA for _ in range(2)],
        ],
        compiler_params=pltpu.CompilerParams(
            use_tc_tiling_on_sc=False,
            needs_layout_passes=False,
        ),
    )(_sc_body)
    out = fn(table, conn, w)
    return out.reshape(_B, _IN_H, _IN_W)
